# baseline (speedup 1.0000x reference)
import jax
import jax.numpy as jnp
from jax import lax
from jax.experimental import pallas as pl
from jax.experimental.pallas import tpu as pltpu
from jax.experimental.pallas import tpu_sc as plsc

B, S, D = 2, 2048, 1024
E, F = 8, 2048
T = B * S
NPAIR = 2 * T
TILE = 256
NT = NPAIR // TILE + E
NPAD = NT * TILE
L = 16
NW = 32
TPW = T // (NW // 2)
NVREG = NPAIR // L
RB = 32

_NEG_INF = float("-inf")


def _router_body(x_ref, wr_ref, e0_ref, e1_ref, w0_ref, w1_ref):
    xt = x_ref[...]
    logits = jnp.dot(xt, wr_ref[...], preferred_element_type=jnp.float32)
    rt = logits.shape[0]
    col = lax.broadcasted_iota(jnp.int32, (rt, 128), 1)
    logits = jnp.where(col < E, logits, _NEG_INF)
    v0 = jnp.max(logits, axis=1, keepdims=True)
    i0 = jnp.min(jnp.where(logits == v0, col, 128), axis=1, keepdims=True)
    l2 = jnp.where(col == i0, _NEG_INF, logits)
    v1 = jnp.max(l2, axis=1, keepdims=True)
    i1 = jnp.min(jnp.where(l2 == v1, col, 128), axis=1, keepdims=True)
    d = jnp.exp(v1 - v0)
    w0 = 1.0 / (1.0 + d)
    w1 = d / (1.0 + d)
    e0_ref[...] = jnp.broadcast_to(i0, (rt, 8))
    e1_ref[...] = jnp.broadcast_to(i1, (rt, 8))
    w0_ref[...] = jnp.broadcast_to(w0, (rt, 8))
    w1_ref[...] = jnp.broadcast_to(w1, (rt, 8))


def _router(x2d, wr_pad):
    rt = 512
    grid = (T // rt,)
    return pl.pallas_call(
        _router_body,
        grid=grid,
        in_specs=[
            pl.BlockSpec((rt, D), lambda i: (i, 0)),
            pl.BlockSpec((D, 128), lambda i: (0, 0)),
        ],
        out_specs=[
            pl.BlockSpec((rt, 8), lambda i: (i, 0)),
            pl.BlockSpec((rt, 8), lambda i: (i, 0)),
            pl.BlockSpec((rt, 8), lambda i: (i, 0)),
            pl.BlockSpec((rt, 8), lambda i: (i, 0)),
        ],
        out_shape=[
            jax.ShapeDtypeStruct((T, 8), jnp.int32),
            jax.ShapeDtypeStruct((T, 8), jnp.int32),
            jax.ShapeDtypeStruct((T, 8), jnp.float32),
            jax.ShapeDtypeStruct((T, 8), jnp.float32),
        ],
    )(x2d, wr_pad)


def _route_body(e0_hbm, e1_hbm, w0_hbm, w1_hbm, x_hbm, pos0_hbm, pos1_hbm,
                xg_hbm, rw_hbm, te_hbm, nu_hbm,
                eall, pb0, pb1, pb2, pb3, pb4, pb5, pb6, pb7, wvals,
                wrowA, wrowB, xrowA, xrowB, tebuf, nubuf,
                isemA, isemB, xsemA, xsemB, wsemA, wsemB):
    cid = lax.axis_index("c")
    sid = lax.axis_index("s")
    wid = sid * 2 + cid
    lane = jnp.arange(L, dtype=jnp.int32)

    pltpu.sync_copy(e0_hbm, eall.at[pl.ds(0, T)])
    pltpu.sync_copy(e1_hbm, eall.at[pl.ds(T, T)])
    tbase = (wid % 16) * TPW

    @pl.when(wid < 16)
    def _():
        pltpu.sync_copy(w0_hbm.at[pl.ds(tbase, TPW)], wvals)

    @pl.when(wid >= 16)
    def _():
        pltpu.sync_copy(w1_hbm.at[pl.ds(tbase, TPW)], wvals)

    xrows = (xrowA, xrowB)
    isems = (isemA, isemB)
    incp = [None, None]
    incp[0] = pltpu.async_copy(x_hbm.at[pl.ds(tbase, RB)], xrows[0], isems[0])
    incp[1] = pltpu.async_copy(x_hbm.at[pl.ds(tbase + RB, RB)], xrows[1],
                               isems[1])

    mstart = wid * (TPW // L)
    zeros = jnp.zeros((L,), jnp.int32)

    def scan_body(i, carry):
        counts, pre = carry
        pre = jnp.where(i == mstart, counts, pre)
        eid = eall[pl.ds(i * L, L)]
        for e in range(E):
            m = eid == e
            p = plsc.all_reduce_population_count(m)
            counts = counts + jnp.where(lane == e, p, 0)
        return counts, pre

    tot, pre = lax.fori_loop(0, NVREG, scan_body, (zeros, zeros), unroll=8)

    padded = ((tot + (TILE - 1)) // TILE) * TILE
    incl = plsc.cumsum(padded)
    off = incl - padded
    basevec = off + pre

    def lane_scalar(vec, e):
        return jnp.sum(jnp.where(lane == e, vec, 0))

    bases = [lane_scalar(basevec, e) for e in range(E)]

    pair0 = wid * TPW
    for v in range(TPW // L):
        eid = eall[pl.ds(pair0 + v * L, L)]
        pos = zeros
        for e in range(E):
            m = eid == e
            mi = m.astype(jnp.int32)
            rank = plsc.cumsum(mi) - 1
            pos = jnp.where(m, rank + bases[e], pos)
            bases[e] = bases[e] + jnp.sum(mi)
        pbufs = (pb0, pb1, pb2, pb3, pb4, pb5, pb6, pb7)
        pbufs[v // (RB // L)][pl.ds((v % (RB // L)) * L, L)] = pos

    for b, pb in enumerate((pb0, pb1, pb2, pb3, pb4, pb5, pb6, pb7)):
        @pl.when(wid < 16)
        def _(b=b, pb=pb):
            pltpu.sync_copy(pb, pos0_hbm.at[wid * (TPW // RB) + b])

        @pl.when(wid >= 16)
        def _(b=b, pb=pb):
            pltpu.sync_copy(
                pb, pos1_hbm.at[(wid - 16) * (TPW // RB) + b])

    nb = TPW // RB
    pbufs = (pb0, pb1, pb2, pb3, pb4, pb5, pb6, pb7)
    wrows = (wrowA, wrowB)
    xsems = (xsemA, xsemB)
    wsems = (wsemA, wsemB)
    xcp = [None, None]
    wcp = [None, None]
    for b in range(nb):
        s = b % 2
        incp[s].wait()
        for g in range(RB // L):
            wv = wvals[pl.ds(b * RB + g * L, L)]
            for r in range(L):
                wrows[s][g * L + r, pl.ds(0, L)] = jnp.broadcast_to(wv[r],
                                                                    (L,))
        xcp[s] = pltpu.async_copy(xrows[s], xg_hbm.at[pbufs[b]], xsems[s])
        wcp[s] = pltpu.async_copy(wrows[s], rw_hbm.at[pbufs[b]], wsems[s])
        if b + 2 < nb:
            xcp[s].wait()
            wcp[s].wait()
            incp[s] = pltpu.async_copy(
                x_hbm.at[pl.ds(tbase + (b + 2) * RB, RB)], xrows[s], isems[s])
    xcp[0].wait()
    wcp[0].wait()
    xcp[1].wait()
    wcp[1].wait()

    @pl.when(wid == 0)
    def _():
        total_padded = jnp.sum(padded)
        n_used = total_padded // TILE
        last_e = jnp.max(jnp.where(padded > 0, lane, -1))
        offs = [lane_scalar(off, e) for e in range(E)]
        for vi in range(4):
            ts = (jnp.arange(L, dtype=jnp.int32) + vi * L) * TILE
            te = jnp.full((L,), -1, jnp.int32)
            for e in range(E):
                te = te + jnp.where(ts >= offs[e], 1, 0)
            te = jnp.where(ts >= total_padded, last_e, te)
            tebuf[pl.ds(vi * L, L)] = te
        nubuf[...] = jnp.broadcast_to(n_used, (L,)).astype(jnp.int32)
        pltpu.sync_copy(tebuf, te_hbm)
        pltpu.sync_copy(nubuf, nu_hbm)


def _route(e0, e1, w0, w1, x2d):
    mesh = plsc.VectorSubcoreMesh(core_axis_name="c", subcore_axis_name="s")
    k = pl.kernel(
        _route_body,
        out_type=[
            jax.ShapeDtypeStruct((T // RB, RB), jnp.int32),
            jax.ShapeDtypeStruct((T // RB, RB), jnp.int32),
            jax.ShapeDtypeStruct((NPAD, D), jnp.float32),
            jax.ShapeDtypeStruct((NPAD, 128), jnp.float32),
            jax.ShapeDtypeStruct((64,), jnp.int32),
            jax.ShapeDtypeStruct((L,), jnp.int32),
        ],
        mesh=mesh,
        compiler_params=pltpu.CompilerParams(needs_layout_passes=False),
        scratch_types=(
            [pltpu.VMEM((NPAIR,), jnp.int32)]
            + [pltpu.VMEM((RB,), jnp.int32)] * 8
            + [pltpu.VMEM((TPW,), jnp.float32)]
            + [pltpu.VMEM((RB, 128), jnp.float32)] * 2
            + [pltpu.VMEM((RB, D), jnp.float32)] * 2
            + [pltpu.VMEM((64,), jnp.int32),
               pltpu.VMEM((L,), jnp.int32)]
            + [pltpu.SemaphoreType.DMA] * 6
        ),
    )
    return k(e0, e1, w0, w1, x2d)


def _ffn_body(te_ref, nu_ref, xg_ref, w1_ref, w2_ref, rw_ref, o_ref):
    i = pl.program_id(0)

    @pl.when(i < nu_ref[0])
    def _():
        xt = xg_ref[...]
        h = jnp.dot(xt, w1_ref[0], preferred_element_type=jnp.float32)
        h = jax.nn.gelu(h)
        y = jnp.dot(h, w2_ref[0], preferred_element_type=jnp.float32)
        o_ref[...] = y * rw_ref[...][:, 0:1]


def _ffn(te, nu, xg, w1, w2, rw):
    grid_spec = pltpu.PrefetchScalarGridSpec(
        num_scalar_prefetch=2,
        grid=(NT,),
        in_specs=[
            pl.BlockSpec((TILE, D), lambda i, te, nu: (i, 0)),
            pl.BlockSpec((1, D, F), lambda i, te, nu: (te[i], 0, 0)),
            pl.BlockSpec((1, F, D), lambda i, te, nu: (te[i], 0, 0)),
            pl.BlockSpec((TILE, 128), lambda i, te, nu: (i, 0)),
        ],
        out_specs=pl.BlockSpec((TILE, D), lambda i, te, nu: (i, 0)),
    )
    return pl.pallas_call(
        _ffn_body,
        grid_spec=grid_spec,
        out_shape=jax.ShapeDtypeStruct((NPAD, D), jnp.float32),
        compiler_params=pltpu.CompilerParams(
            dimension_semantics=("arbitrary",),
        ),
    )(te, nu, xg, w1, w2, rw)


CB = 16


def _combine_body(yg_hbm, p0_hbm, p1_hbm, out_hbm, p0b, p1b,
                  y0A, y0B, y1A, y1B,
                  g0A, g0B, g1A, g1B, osA, osB):
    cid = lax.axis_index("c")
    sid = lax.axis_index("s")
    wid = sid * 2 + cid
    tpw = T // NW
    tbase = wid * tpw
    nb = tpw // CB

    pltpu.sync_copy(p0_hbm.at[pl.ds(wid * (tpw // RB), tpw // RB)], p0b)
    pltpu.sync_copy(p1_hbm.at[pl.ds(wid * (tpw // RB), tpw // RB)], p1b)

    y0s = (y0A, y0B)
    y1s = (y1A, y1B)
    g0s = (g0A, g0B)
    g1s = (g1A, g1B)
    oss = (osA, osB)

    def idx(pb, b):
        return pb.at[b // 2, pl.ds((b % 2) * CB, CB)]

    cp0 = [None, None]
    cp1 = [None, None]
    wcp = [None, None]
    cp0[0] = pltpu.async_copy(yg_hbm.at[idx(p0b, 0)], y0s[0], g0s[0])
    cp1[0] = pltpu.async_copy(yg_hbm.at[idx(p1b, 0)], y1s[0], g1s[0])
    for b in range(nb):
        s = b % 2
        n = (b + 1) % 2
        if b + 1 < nb:
            if wcp[n] is not None:
                wcp[n].wait()
            cp0[n] = pltpu.async_copy(yg_hbm.at[idx(p0b, b + 1)], y0s[n],
                                      g0s[n])
            cp1[n] = pltpu.async_copy(yg_hbm.at[idx(p1b, b + 1)], y1s[n],
                                      g1s[n])
        cp0[s].wait()
        cp1[s].wait()

        def row_add(r, _, s=s):
            def d_add(dd, __):
                y0s[s][r, pl.ds(dd * L, L)] = (y0s[s][r, pl.ds(dd * L, L)]
                                               + y1s[s][r, pl.ds(dd * L, L)])
                return __
            return lax.fori_loop(0, D // L, d_add, _, unroll=16)

        lax.fori_loop(0, CB, row_add, 0, unroll=8)
        wcp[s] = pltpu.async_copy(y0s[s], out_hbm.at[pl.ds(tbase + b * CB,
                                                           CB)], oss[s])
    wcp[0].wait()
    wcp[1].wait()


def _combine(yg, pos0, pos1):
    mesh = plsc.VectorSubcoreMesh(core_axis_name="c", subcore_axis_name="s")
    nb = (T // NW) // RB
    k = pl.kernel(
        _combine_body,
        out_type=jax.ShapeDtypeStruct((T, D), jnp.float32),
        mesh=mesh,
        compiler_params=pltpu.CompilerParams(needs_layout_passes=False),
        scratch_types=(
            [pltpu.VMEM((nb, RB), jnp.int32)] * 2
            + [pltpu.VMEM((CB, D), jnp.float32)] * 4
            + [pltpu.SemaphoreType.DMA] * 6
        ),
    )
    return k(yg, pos0, pos1)


@jax.jit
def kernel(x, Wr, W1, W2):
    x2d = x.reshape(T, D)
    wr_pad = jnp.pad(Wr, ((0, 0), (0, 128 - E)))
    e0b, e1b, w0b, w1b = _router(x2d, wr_pad)
    e0 = e0b[:, 0]
    e1 = e1b[:, 0]
    w0 = w0b[:, 0]
    w1 = w1b[:, 0]
    pos0, pos1, xg, rw, te, nu = _route(e0, e1, w0, w1, x2d)
    yg = _ffn(te, nu, xg, W1, W2, rw)
    out = _combine(yg, pos0, pos1)
    return out.reshape(B, S, D)


# speedup vs baseline: 1.0609x; 1.0609x over previous
import jax
import jax.numpy as jnp
from jax import lax
from jax.experimental import pallas as pl
from jax.experimental.pallas import tpu as pltpu
from jax.experimental.pallas import tpu_sc as plsc

B, S, D = 2, 2048, 1024
E, F = 8, 2048
T = B * S
NPAIR = 2 * T
TILE = 256
NT = NPAIR // TILE + E
NPAD = NT * TILE
L = 16
NW = 32
TPW = T // (NW // 2)
NVREG = NPAIR // L
RB = 32

_NEG_INF = float("-inf")


def _router_body(x_ref, wr_ref, e0_ref, e1_ref, w0_ref, w1_ref):
    xt = x_ref[...]
    logits = jnp.dot(xt, wr_ref[...], preferred_element_type=jnp.float32)
    rt = logits.shape[0]
    col = lax.broadcasted_iota(jnp.int32, (rt, 128), 1)
    logits = jnp.where(col < E, logits, _NEG_INF)
    v0 = jnp.max(logits, axis=1, keepdims=True)
    i0 = jnp.min(jnp.where(logits == v0, col, 128), axis=1, keepdims=True)
    l2 = jnp.where(col == i0, _NEG_INF, logits)
    v1 = jnp.max(l2, axis=1, keepdims=True)
    i1 = jnp.min(jnp.where(l2 == v1, col, 128), axis=1, keepdims=True)
    d = jnp.exp(v1 - v0)
    w0 = 1.0 / (1.0 + d)
    w1 = d / (1.0 + d)
    e0_ref[...] = i0.reshape(1, 1, rt)
    e1_ref[...] = i1.reshape(1, 1, rt)
    w0_ref[...] = w0.reshape(1, 1, rt)
    w1_ref[...] = w1.reshape(1, 1, rt)


def _router(x2d, wr_pad):
    rt = 512
    grid = (T // rt,)
    return pl.pallas_call(
        _router_body,
        grid=grid,
        in_specs=[
            pl.BlockSpec((rt, D), lambda i: (i, 0)),
            pl.BlockSpec((D, 128), lambda i: (0, 0)),
        ],
        out_specs=[
            pl.BlockSpec((1, 1, rt), lambda i: (i, 0, 0)),
            pl.BlockSpec((1, 1, rt), lambda i: (i, 0, 0)),
            pl.BlockSpec((1, 1, rt), lambda i: (i, 0, 0)),
            pl.BlockSpec((1, 1, rt), lambda i: (i, 0, 0)),
        ],
        out_shape=[
            jax.ShapeDtypeStruct((T // 512, 1, 512), jnp.int32),
            jax.ShapeDtypeStruct((T // 512, 1, 512), jnp.int32),
            jax.ShapeDtypeStruct((T // 512, 1, 512), jnp.float32),
            jax.ShapeDtypeStruct((T // 512, 1, 512), jnp.float32),
        ],
    )(x2d, wr_pad)


def _route_body(e0_hbm, e1_hbm, w0_hbm, w1_hbm, x_hbm, pos0_hbm, pos1_hbm,
                xg_hbm, rw_hbm, te_hbm, nu_hbm,
                eall, pb0, pb1, pb2, pb3, pb4, pb5, pb6, pb7, wvals,
                wrowA, wrowB, xrowA, xrowB, tebuf, nubuf,
                isemA, isemB, xsemA, xsemB, wsemA, wsemB):
    cid = lax.axis_index("c")
    sid = lax.axis_index("s")
    wid = sid * 2 + cid
    lane = jnp.arange(L, dtype=jnp.int32)

    for r in range(T // 512):
        pltpu.sync_copy(e0_hbm.at[r, 0], eall.at[pl.ds(r * 512, 512)])
        pltpu.sync_copy(e1_hbm.at[r, 0], eall.at[pl.ds(T + r * 512, 512)])
    tbase = (wid % 16) * TPW
    trow = tbase // 512
    toff = tbase % 512

    @pl.when(wid < 16)
    def _():
        pltpu.sync_copy(w0_hbm.at[trow, 0, pl.ds(toff, TPW)], wvals)

    @pl.when(wid >= 16)
    def _():
        pltpu.sync_copy(w1_hbm.at[trow, 0, pl.ds(toff, TPW)], wvals)

    xrows = (xrowA, xrowB)
    isems = (isemA, isemB)
    incp = [None, None]
    incp[0] = pltpu.async_copy(x_hbm.at[pl.ds(tbase, RB)], xrows[0], isems[0])
    incp[1] = pltpu.async_copy(x_hbm.at[pl.ds(tbase + RB, RB)], xrows[1],
                               isems[1])

    mstart = wid * (TPW // L)
    zeros = jnp.zeros((L,), jnp.int32)

    def scan_body(i, carry):
        counts, pre = carry
        pre = jnp.where(i == mstart, counts, pre)
        eid = eall[pl.ds(i * L, L)]
        for e in range(E):
            m = eid == e
            p = plsc.all_reduce_population_count(m)
            counts = counts + jnp.where(lane == e, p, 0)
        return counts, pre

    tot, pre = lax.fori_loop(0, NVREG, scan_body, (zeros, zeros), unroll=4)

    padded = ((tot + (TILE - 1)) // TILE) * TILE
    incl = plsc.cumsum(padded)
    off = incl - padded
    basevec = off + pre

    def lane_scalar(vec, e):
        return jnp.sum(jnp.where(lane == e, vec, 0))

    bases = [lane_scalar(basevec, e) for e in range(E)]

    pair0 = wid * TPW
    for v in range(TPW // L):
        eid = eall[pl.ds(pair0 + v * L, L)]
        pos = zeros
        for e in range(E):
            m = eid == e
            mi = m.astype(jnp.int32)
            rank = plsc.cumsum(mi) - 1
            pos = jnp.where(m, rank + bases[e], pos)
            bases[e] = bases[e] + jnp.sum(mi)
        pbufs = (pb0, pb1, pb2, pb3, pb4, pb5, pb6, pb7)
        pbufs[v // (RB // L)][pl.ds((v % (RB // L)) * L, L)] = pos

    for b, pb in enumerate((pb0, pb1, pb2, pb3, pb4, pb5, pb6, pb7)):
        @pl.when(wid < 16)
        def _(b=b, pb=pb):
            pltpu.sync_copy(pb, pos0_hbm.at[wid * (TPW // RB) + b])

        @pl.when(wid >= 16)
        def _(b=b, pb=pb):
            pltpu.sync_copy(
                pb, pos1_hbm.at[(wid - 16) * (TPW // RB) + b])

    nb = TPW // RB
    pbufs = (pb0, pb1, pb2, pb3, pb4, pb5, pb6, pb7)
    wrows = (wrowA, wrowB)
    xsems = (xsemA, xsemB)
    wsems = (wsemA, wsemB)
    xcp = [None, None]
    wcp = [None, None]
    for b in range(nb):
        s = b % 2
        incp[s].wait()
        for g in range(RB // L):
            wv = wvals[pl.ds(b * RB + g * L, L)]
            for r in range(L):
                wrows[s][g * L + r, pl.ds(0, L)] = jnp.broadcast_to(wv[r],
                                                                    (L,))
        xcp[s] = pltpu.async_copy(xrows[s], xg_hbm.at[pbufs[b]], xsems[s])
        wcp[s] = pltpu.async_copy(wrows[s], rw_hbm.at[pbufs[b]], wsems[s])
        if b + 2 < nb:
            xcp[s].wait()
            wcp[s].wait()
            incp[s] = pltpu.async_copy(
                x_hbm.at[pl.ds(tbase + (b + 2) * RB, RB)], xrows[s], isems[s])
    xcp[0].wait()
    wcp[0].wait()
    xcp[1].wait()
    wcp[1].wait()

    @pl.when(wid == 0)
    def _():
        total_padded = jnp.sum(padded)
        n_used = total_padded // TILE
        last_e = jnp.max(jnp.where(padded > 0, lane, -1))
        offs = [lane_scalar(off, e) for e in range(E)]
        for vi in range(4):
            ts = (jnp.arange(L, dtype=jnp.int32) + vi * L) * TILE
            te = jnp.full((L,), -1, jnp.int32)
            for e in range(E):
                te = te + jnp.where(ts >= offs[e], 1, 0)
            te = jnp.where(ts >= total_padded, last_e, te)
            tebuf[pl.ds(vi * L, L)] = te
        nubuf[...] = jnp.broadcast_to(n_used, (L,)).astype(jnp.int32)
        pltpu.sync_copy(tebuf, te_hbm)
        pltpu.sync_copy(nubuf, nu_hbm)


def _route(e0, e1, w0, w1, x2d):
    mesh = plsc.VectorSubcoreMesh(core_axis_name="c", subcore_axis_name="s")
    k = pl.kernel(
        _route_body,
        out_type=[
            jax.ShapeDtypeStruct((T // RB, RB), jnp.int32),
            jax.ShapeDtypeStruct((T // RB, RB), jnp.int32),
            jax.ShapeDtypeStruct((NPAD, D), jnp.float32),
            jax.ShapeDtypeStruct((NPAD, 128), jnp.float32),
            jax.ShapeDtypeStruct((64,), jnp.int32),
            jax.ShapeDtypeStruct((L,), jnp.int32),
        ],
        mesh=mesh,
        compiler_params=pltpu.CompilerParams(needs_layout_passes=False),
        scratch_types=(
            [pltpu.VMEM((NPAIR,), jnp.int32)]
            + [pltpu.VMEM((RB,), jnp.int32)] * 8
            + [pltpu.VMEM((TPW,), jnp.float32)]
            + [pltpu.VMEM((RB, 128), jnp.float32)] * 2
            + [pltpu.VMEM((RB, D), jnp.float32)] * 2
            + [pltpu.VMEM((64,), jnp.int32),
               pltpu.VMEM((L,), jnp.int32)]
            + [pltpu.SemaphoreType.DMA] * 6
        ),
    )
    return k(e0, e1, w0, w1, x2d)


def _ffn_body(te_ref, nu_ref, xg_ref, w1_ref, w2_ref, rw_ref, o_ref):
    i = pl.program_id(0)

    @pl.when(i < nu_ref[0])
    def _():
        xt = xg_ref[...]
        h = jnp.dot(xt, w1_ref[0], preferred_element_type=jnp.float32)
        h = jax.nn.gelu(h)
        y = jnp.dot(h, w2_ref[0], preferred_element_type=jnp.float32)
        o_ref[...] = y * rw_ref[...][:, 0:1]


def _ffn(te, nu, xg, w1, w2, rw):
    grid_spec = pltpu.PrefetchScalarGridSpec(
        num_scalar_prefetch=2,
        grid=(NT,),
        in_specs=[
            pl.BlockSpec((TILE, D), lambda i, te, nu: (i, 0)),
            pl.BlockSpec((1, D, F), lambda i, te, nu: (te[i], 0, 0)),
            pl.BlockSpec((1, F, D), lambda i, te, nu: (te[i], 0, 0)),
            pl.BlockSpec((TILE, 128), lambda i, te, nu: (i, 0)),
        ],
        out_specs=pl.BlockSpec((TILE, D), lambda i, te, nu: (i, 0)),
    )
    return pl.pallas_call(
        _ffn_body,
        grid_spec=grid_spec,
        out_shape=jax.ShapeDtypeStruct((NPAD, D), jnp.float32),
        compiler_params=pltpu.CompilerParams(
            dimension_semantics=("arbitrary",),
        ),
    )(te, nu, xg, w1, w2, rw)


CB = 16


def _combine_body(yg_hbm, p0_hbm, p1_hbm, out_hbm, p0b, p1b,
                  y0A, y0B, y1A, y1B,
                  g0A, g0B, g1A, g1B, osA, osB):
    cid = lax.axis_index("c")
    sid = lax.axis_index("s")
    wid = sid * 2 + cid
    tpw = T // NW
    tbase = wid * tpw
    nb = tpw // CB

    pltpu.sync_copy(p0_hbm.at[pl.ds(wid * (tpw // RB), tpw // RB)], p0b)
    pltpu.sync_copy(p1_hbm.at[pl.ds(wid * (tpw // RB), tpw // RB)], p1b)

    y0s = (y0A, y0B)
    y1s = (y1A, y1B)
    g0s = (g0A, g0B)
    g1s = (g1A, g1B)
    oss = (osA, osB)

    def idx(pb, b):
        return pb.at[b // 2, pl.ds((b % 2) * CB, CB)]

    cp0 = [None, None]
    cp1 = [None, None]
    wcp = [None, None]
    cp0[0] = pltpu.async_copy(yg_hbm.at[idx(p0b, 0)], y0s[0], g0s[0])
    cp1[0] = pltpu.async_copy(yg_hbm.at[idx(p1b, 0)], y1s[0], g1s[0])
    for b in range(nb):
        s = b % 2
        n = (b + 1) % 2
        if b + 1 < nb:
            if wcp[n] is not None:
                wcp[n].wait()
            cp0[n] = pltpu.async_copy(yg_hbm.at[idx(p0b, b + 1)], y0s[n],
                                      g0s[n])
            cp1[n] = pltpu.async_copy(yg_hbm.at[idx(p1b, b + 1)], y1s[n],
                                      g1s[n])
        cp0[s].wait()
        cp1[s].wait()

        def row_add(r, _, s=s):
            def d_add(dd, __):
                y0s[s][r, pl.ds(dd * L, L)] = (y0s[s][r, pl.ds(dd * L, L)]
                                               + y1s[s][r, pl.ds(dd * L, L)])
                return __
            return lax.fori_loop(0, D // L, d_add, _, unroll=16)

        lax.fori_loop(0, CB, row_add, 0, unroll=4)
        wcp[s] = pltpu.async_copy(y0s[s], out_hbm.at[pl.ds(tbase + b * CB,
                                                           CB)], oss[s])
    wcp[0].wait()
    wcp[1].wait()


def _combine(yg, pos0, pos1):
    mesh = plsc.VectorSubcoreMesh(core_axis_name="c", subcore_axis_name="s")
    nb = (T // NW) // RB
    k = pl.kernel(
        _combine_body,
        out_type=jax.ShapeDtypeStruct((T, D), jnp.float32),
        mesh=mesh,
        compiler_params=pltpu.CompilerParams(needs_layout_passes=False),
        scratch_types=(
            [pltpu.VMEM((nb, RB), jnp.int32)] * 2
            + [pltpu.VMEM((CB, D), jnp.float32)] * 4
            + [pltpu.SemaphoreType.DMA] * 6
        ),
    )
    return k(yg, pos0, pos1)


@jax.jit
def kernel(x, Wr, W1, W2):
    x2d = x.reshape(T, D)
    wr_pad = jnp.pad(Wr, ((0, 0), (0, 128 - E)))
    e0, e1, w0, w1 = _router(x2d, wr_pad)
    pos0, pos1, xg, rw, te, nu = _route(e0, e1, w0, w1, x2d)
    yg = _ffn(te, nu, xg, W1, W2, rw)
    out = _combine(yg, pos0, pos1)
    return out.reshape(B, S, D)


# speedup vs baseline: 1.1284x; 1.0637x over previous
import jax
import jax.numpy as jnp
from jax import lax
from jax.experimental import pallas as pl
from jax.experimental.pallas import tpu as pltpu
from jax.experimental.pallas import tpu_sc as plsc

B, S, D = 2, 2048, 1024
E, F = 8, 2048
T = B * S
NPAIR = 2 * T
TILE = 256
NT = NPAIR // TILE + E
NPAD = NT * TILE
L = 16
NW = 32
TPW = T // (NW // 2)
NVREG = NPAIR // L
RB = 32

_NEG_INF = float("-inf")


def _router_body(x_ref, wr_ref, e0_ref, e1_ref, w0_ref, w1_ref):
    xt = x_ref[...]
    logits = jnp.dot(xt, wr_ref[...], preferred_element_type=jnp.float32)
    rt = logits.shape[0]
    col = lax.broadcasted_iota(jnp.int32, (rt, 128), 1)
    logits = jnp.where(col < E, logits, _NEG_INF)
    v0 = jnp.max(logits, axis=1, keepdims=True)
    i0 = jnp.min(jnp.where(logits == v0, col, 128), axis=1, keepdims=True)
    l2 = jnp.where(col == i0, _NEG_INF, logits)
    v1 = jnp.max(l2, axis=1, keepdims=True)
    i1 = jnp.min(jnp.where(l2 == v1, col, 128), axis=1, keepdims=True)
    d = jnp.exp(v1 - v0)
    w0 = 1.0 / (1.0 + d)
    w1 = d / (1.0 + d)
    e0_ref[...] = jnp.broadcast_to(i0, (rt, 8))
    e1_ref[...] = jnp.broadcast_to(i1, (rt, 8))
    w0_ref[...] = jnp.broadcast_to(w0, (rt, 8))
    w1_ref[...] = jnp.broadcast_to(w1, (rt, 8))


def _router(x2d, wr_pad):
    rt = 512
    grid = (T // rt,)
    return pl.pallas_call(
        _router_body,
        grid=grid,
        in_specs=[
            pl.BlockSpec((rt, D), lambda i: (i, 0)),
            pl.BlockSpec((D, 128), lambda i: (0, 0)),
        ],
        out_specs=[
            pl.BlockSpec((rt, 8), lambda i: (i, 0)),
            pl.BlockSpec((rt, 8), lambda i: (i, 0)),
            pl.BlockSpec((rt, 8), lambda i: (i, 0)),
            pl.BlockSpec((rt, 8), lambda i: (i, 0)),
        ],
        out_shape=[
            jax.ShapeDtypeStruct((T, 8), jnp.int32),
            jax.ShapeDtypeStruct((T, 8), jnp.int32),
            jax.ShapeDtypeStruct((T, 8), jnp.float32),
            jax.ShapeDtypeStruct((T, 8), jnp.float32),
        ],
    )(x2d, wr_pad)


def _route_body(e0_hbm, e1_hbm, w0_hbm, w1_hbm, x_hbm, pos0_hbm, pos1_hbm,
                xg_hbm, rw_hbm, te_hbm, nu_hbm,
                eall, pb0, pb1, pb2, pb3, pb4, pb5, pb6, pb7, wvals,
                wrowA, wrowB, xrowA, xrowB, tebuf, nubuf,
                isemA, isemB, xsemA, xsemB, wsemA, wsemB):
    cid = lax.axis_index("c")
    sid = lax.axis_index("s")
    wid = sid * 2 + cid
    lane = jnp.arange(L, dtype=jnp.int32)

    pltpu.sync_copy(e0_hbm, eall.at[pl.ds(0, T)])
    pltpu.sync_copy(e1_hbm, eall.at[pl.ds(T, T)])
    tbase = (wid % 16) * TPW

    @pl.when(wid < 16)
    def _():
        pltpu.sync_copy(w0_hbm.at[pl.ds(tbase, TPW)], wvals)

    @pl.when(wid >= 16)
    def _():
        pltpu.sync_copy(w1_hbm.at[pl.ds(tbase, TPW)], wvals)

    xrows = (xrowA, xrowB)
    isems = (isemA, isemB)
    incp = [None, None]
    incp[0] = pltpu.async_copy(x_hbm.at[pl.ds(tbase, RB)], xrows[0], isems[0])
    incp[1] = pltpu.async_copy(x_hbm.at[pl.ds(tbase + RB, RB)], xrows[1],
                               isems[1])

    mstart = wid * (TPW // L)
    zeros = jnp.zeros((L,), jnp.int32)

    def scan_body(i, carry):
        counts, pre = carry
        pre = jnp.where(i == mstart, counts, pre)
        eid = eall[pl.ds(i * L, L)]
        for e in range(E):
            m = eid == e
            p = plsc.all_reduce_population_count(m)
            counts = counts + jnp.where(lane == e, p, 0)
        return counts, pre

    tot, pre = lax.fori_loop(0, NVREG, scan_body, (zeros, zeros), unroll=4)

    padded = ((tot + (TILE - 1)) // TILE) * TILE
    incl = plsc.cumsum(padded)
    off = incl - padded
    basevec = off + pre

    def lane_scalar(vec, e):
        return jnp.sum(jnp.where(lane == e, vec, 0))

    bases = [lane_scalar(basevec, e) for e in range(E)]

    pair0 = wid * TPW
    for v in range(TPW // L):
        eid = eall[pl.ds(pair0 + v * L, L)]
        pos = zeros
        for e in range(E):
            m = eid == e
            mi = m.astype(jnp.int32)
            rank = plsc.cumsum(mi) - 1
            pos = jnp.where(m, rank + bases[e], pos)
            bases[e] = bases[e] + jnp.sum(mi)
        pbufs = (pb0, pb1, pb2, pb3, pb4, pb5, pb6, pb7)
        pbufs[v // (RB // L)][pl.ds((v % (RB // L)) * L, L)] = pos

    for b, pb in enumerate((pb0, pb1, pb2, pb3, pb4, pb5, pb6, pb7)):
        @pl.when(wid < 16)
        def _(b=b, pb=pb):
            pltpu.sync_copy(pb, pos0_hbm.at[wid * (TPW // RB) + b])

        @pl.when(wid >= 16)
        def _(b=b, pb=pb):
            pltpu.sync_copy(
                pb, pos1_hbm.at[(wid - 16) * (TPW // RB) + b])

    nb = TPW // RB
    pbufs = (pb0, pb1, pb2, pb3, pb4, pb5, pb6, pb7)
    wrows = (wrowA, wrowB)
    xsems = (xsemA, xsemB)
    wsems = (wsemA, wsemB)
    xcp = [None, None]
    wcp = [None, None]
    for b in range(nb):
        s = b % 2
        incp[s].wait()
        for g in range(RB // L):
            wv = wvals[pl.ds(b * RB + g * L, L)]
            for r in range(L):
                wrows[s][g * L + r, pl.ds(0, L)] = jnp.broadcast_to(wv[r],
                                                                    (L,))
        xcp[s] = pltpu.async_copy(xrows[s], xg_hbm.at[pbufs[b]], xsems[s])
        wcp[s] = pltpu.async_copy(wrows[s], rw_hbm.at[pbufs[b]], wsems[s])
        if b + 2 < nb:
            xcp[s].wait()
            wcp[s].wait()
            incp[s] = pltpu.async_copy(
                x_hbm.at[pl.ds(tbase + (b + 2) * RB, RB)], xrows[s], isems[s])
    xcp[0].wait()
    wcp[0].wait()
    xcp[1].wait()
    wcp[1].wait()

    @pl.when(wid == 0)
    def _():
        total_padded = jnp.sum(padded)
        n_used = total_padded // TILE
        last_e = jnp.max(jnp.where(padded > 0, lane, -1))
        offs = [lane_scalar(off, e) for e in range(E)]
        for vi in range(4):
            ts = (jnp.arange(L, dtype=jnp.int32) + vi * L) * TILE
            te = jnp.full((L,), -1, jnp.int32)
            for e in range(E):
                te = te + jnp.where(ts >= offs[e], 1, 0)
            te = jnp.where(ts >= total_padded, last_e, te)
            tebuf[pl.ds(vi * L, L)] = te
        nubuf[...] = jnp.broadcast_to(n_used, (L,)).astype(jnp.int32)
        pltpu.sync_copy(tebuf, te_hbm)
        pltpu.sync_copy(nubuf, nu_hbm)


def _route(e0, e1, w0, w1, x2d):
    mesh = plsc.VectorSubcoreMesh(core_axis_name="c", subcore_axis_name="s")
    k = pl.kernel(
        _route_body,
        out_type=[
            jax.ShapeDtypeStruct((T // RB, RB), jnp.int32),
            jax.ShapeDtypeStruct((T // RB, RB), jnp.int32),
            jax.ShapeDtypeStruct((NPAD, D), jnp.float32),
            jax.ShapeDtypeStruct((NPAD, 128), jnp.float32),
            jax.ShapeDtypeStruct((64,), jnp.int32),
            jax.ShapeDtypeStruct((L,), jnp.int32),
        ],
        mesh=mesh,
        compiler_params=pltpu.CompilerParams(needs_layout_passes=False),
        scratch_types=(
            [pltpu.VMEM((NPAIR,), jnp.int32)]
            + [pltpu.VMEM((RB,), jnp.int32)] * 8
            + [pltpu.VMEM((TPW,), jnp.float32)]
            + [pltpu.VMEM((RB, 128), jnp.float32)] * 2
            + [pltpu.VMEM((RB, D), jnp.float32)] * 2
            + [pltpu.VMEM((64,), jnp.int32),
               pltpu.VMEM((L,), jnp.int32)]
            + [pltpu.SemaphoreType.DMA] * 6
        ),
    )
    return k(e0, e1, w0, w1, x2d)


def _ffn_body(te_ref, nu_ref, xg_ref, w1_ref, w2_ref, rw_ref, o_ref):
    i = pl.program_id(0)

    @pl.when(i < nu_ref[0])
    def _():
        xt = xg_ref[...]
        h = jnp.dot(xt, w1_ref[0], preferred_element_type=jnp.float32)
        h = jax.nn.gelu(h)
        y = jnp.dot(h, w2_ref[0], preferred_element_type=jnp.float32)
        o_ref[...] = y * rw_ref[...][:, 0:1]


def _ffn(te, nu, xg, w1, w2, rw):
    grid_spec = pltpu.PrefetchScalarGridSpec(
        num_scalar_prefetch=2,
        grid=(NT,),
        in_specs=[
            pl.BlockSpec((TILE, D), lambda i, te, nu: (i, 0)),
            pl.BlockSpec((1, D, F), lambda i, te, nu: (te[i], 0, 0)),
            pl.BlockSpec((1, F, D), lambda i, te, nu: (te[i], 0, 0)),
            pl.BlockSpec((TILE, 128), lambda i, te, nu: (i, 0)),
        ],
        out_specs=pl.BlockSpec((TILE, D), lambda i, te, nu: (i, 0)),
    )
    return pl.pallas_call(
        _ffn_body,
        grid_spec=grid_spec,
        out_shape=jax.ShapeDtypeStruct((NPAD, D), jnp.float32),
        compiler_params=pltpu.CompilerParams(
            dimension_semantics=("arbitrary",),
        ),
    )(te, nu, xg, w1, w2, rw)


CB = 16


def _combine_body(yg_hbm, p0_hbm, p1_hbm, out_hbm, p0b, p1b,
                  y0A, y0B, y1A, y1B,
                  g0A, g0B, g1A, g1B, osA, osB):
    cid = lax.axis_index("c")
    sid = lax.axis_index("s")
    wid = sid * 2 + cid
    tpw = T // NW
    tbase = wid * tpw
    nb = tpw // CB

    pltpu.sync_copy(p0_hbm.at[pl.ds(wid * (tpw // RB), tpw // RB)], p0b)
    pltpu.sync_copy(p1_hbm.at[pl.ds(wid * (tpw // RB), tpw // RB)], p1b)

    y0s = (y0A, y0B)
    y1s = (y1A, y1B)
    g0s = (g0A, g0B)
    g1s = (g1A, g1B)
    oss = (osA, osB)

    def idx(pb, b):
        return pb.at[b // 2, pl.ds((b % 2) * CB, CB)]

    cp0 = [None, None]
    cp1 = [None, None]
    wcp = [None, None]
    cp0[0] = pltpu.async_copy(yg_hbm.at[idx(p0b, 0)], y0s[0], g0s[0])
    cp1[0] = pltpu.async_copy(yg_hbm.at[idx(p1b, 0)], y1s[0], g1s[0])
    for b in range(nb):
        s = b % 2
        n = (b + 1) % 2
        if b + 1 < nb:
            if wcp[n] is not None:
                wcp[n].wait()
            cp0[n] = pltpu.async_copy(yg_hbm.at[idx(p0b, b + 1)], y0s[n],
                                      g0s[n])
            cp1[n] = pltpu.async_copy(yg_hbm.at[idx(p1b, b + 1)], y1s[n],
                                      g1s[n])
        cp0[s].wait()
        cp1[s].wait()

        def row_add(r, _, s=s):
            def d_add(dd, __):
                y0s[s][r, pl.ds(dd * L, L)] = (y0s[s][r, pl.ds(dd * L, L)]
                                               + y1s[s][r, pl.ds(dd * L, L)])
                return __
            return lax.fori_loop(0, D // L, d_add, _, unroll=16)

        lax.fori_loop(0, CB, row_add, 0, unroll=4)
        wcp[s] = pltpu.async_copy(y0s[s], out_hbm.at[pl.ds(tbase + b * CB,
                                                           CB)], oss[s])
    wcp[0].wait()
    wcp[1].wait()


def _combine(yg, pos0, pos1):
    mesh = plsc.VectorSubcoreMesh(core_axis_name="c", subcore_axis_name="s")
    nb = (T // NW) // RB
    k = pl.kernel(
        _combine_body,
        out_type=jax.ShapeDtypeStruct((T, D), jnp.float32),
        mesh=mesh,
        compiler_params=pltpu.CompilerParams(needs_layout_passes=False),
        scratch_types=(
            [pltpu.VMEM((nb, RB), jnp.int32)] * 2
            + [pltpu.VMEM((CB, D), jnp.float32)] * 4
            + [pltpu.SemaphoreType.DMA] * 6
        ),
    )
    return k(yg, pos0, pos1)


@jax.jit
def kernel(x, Wr, W1, W2):
    x2d = x.reshape(T, D)
    wr_pad = jnp.pad(Wr, ((0, 0), (0, 128 - E)))
    e0b, e1b, w0b, w1b = _router(x2d, wr_pad)
    e0 = e0b[:, 0]
    e1 = e1b[:, 0]
    w0 = w0b[:, 0]
    w1 = w1b[:, 0]
    pos0, pos1, xg, rw, te, nu = _route(e0, e1, w0, w1, x2d)
    yg = _ffn(te, nu, xg, W1, W2, rw)
    out = _combine(yg, pos0, pos1)
    return out.reshape(B, S, D)


# speedup vs baseline: 1.1343x; 1.0053x over previous
import jax
import jax.numpy as jnp
from jax import lax
from jax.experimental import pallas as pl
from jax.experimental.pallas import tpu as pltpu
from jax.experimental.pallas import tpu_sc as plsc

B, S, D = 2, 2048, 1024
E, F = 8, 2048
T = B * S
NPAIR = 2 * T
TILE = 256
NT = NPAIR // TILE + E
NPAD = NT * TILE
L = 16
NW = 32
TPW = T // (NW // 2)
NVREG = NPAIR // L
RB = 32

_NEG_INF = float("-inf")


def _router_body(x_ref, wr_ref, e0_ref, e1_ref, w0_ref, w1_ref):
    xt = x_ref[...]
    logits = jnp.dot(xt, wr_ref[...], preferred_element_type=jnp.float32)
    rt = logits.shape[0]
    col = lax.broadcasted_iota(jnp.int32, (rt, 128), 1)
    logits = jnp.where(col < E, logits, _NEG_INF)
    v0 = jnp.max(logits, axis=1, keepdims=True)
    i0 = jnp.min(jnp.where(logits == v0, col, 128), axis=1, keepdims=True)
    l2 = jnp.where(col == i0, _NEG_INF, logits)
    v1 = jnp.max(l2, axis=1, keepdims=True)
    i1 = jnp.min(jnp.where(l2 == v1, col, 128), axis=1, keepdims=True)
    d = jnp.exp(v1 - v0)
    w0 = 1.0 / (1.0 + d)
    w1 = d / (1.0 + d)
    e0_ref[...] = jnp.broadcast_to(i0, (rt, 8))
    e1_ref[...] = jnp.broadcast_to(i1, (rt, 8))
    w0_ref[...] = jnp.broadcast_to(w0, (rt, 8))
    w1_ref[...] = jnp.broadcast_to(w1, (rt, 8))


def _router(x2d, wr_pad):
    rt = 512
    grid = (T // rt,)
    return pl.pallas_call(
        _router_body,
        grid=grid,
        in_specs=[
            pl.BlockSpec((rt, D), lambda i: (i, 0)),
            pl.BlockSpec((D, 128), lambda i: (0, 0)),
        ],
        out_specs=[
            pl.BlockSpec((rt, 8), lambda i: (i, 0)),
            pl.BlockSpec((rt, 8), lambda i: (i, 0)),
            pl.BlockSpec((rt, 8), lambda i: (i, 0)),
            pl.BlockSpec((rt, 8), lambda i: (i, 0)),
        ],
        out_shape=[
            jax.ShapeDtypeStruct((T, 8), jnp.int32),
            jax.ShapeDtypeStruct((T, 8), jnp.int32),
            jax.ShapeDtypeStruct((T, 8), jnp.float32),
            jax.ShapeDtypeStruct((T, 8), jnp.float32),
        ],
    )(x2d, wr_pad)


def _route_body(e0_hbm, e1_hbm, w0_hbm, w1_hbm, x_hbm, pos0_hbm, pos1_hbm,
                xg_hbm, rw_hbm, te_hbm, nu_hbm,
                eall, pb0, pb1, pb2, pb3, pb4, pb5, pb6, pb7, posflat, wvals,
                wrowA, wrowB, xrowA, xrowB, tebuf, nubuf,
                isemA, isemB, xsemA, xsemB, wsemA, wsemB, psem):
    cid = lax.axis_index("c")
    sid = lax.axis_index("s")
    wid = sid * 2 + cid
    lane = jnp.arange(L, dtype=jnp.int32)

    pltpu.sync_copy(e0_hbm, eall.at[pl.ds(0, T)])
    pltpu.sync_copy(e1_hbm, eall.at[pl.ds(T, T)])
    tbase = (wid % 16) * TPW

    @pl.when(wid < 16)
    def _():
        pltpu.sync_copy(w0_hbm.at[pl.ds(tbase, TPW)], wvals)

    @pl.when(wid >= 16)
    def _():
        pltpu.sync_copy(w1_hbm.at[pl.ds(tbase, TPW)], wvals)

    xrows = (xrowA, xrowB)
    isems = (isemA, isemB)
    incp = [None, None]
    incp[0] = pltpu.async_copy(x_hbm.at[pl.ds(tbase, RB)], xrows[0], isems[0])
    incp[1] = pltpu.async_copy(x_hbm.at[pl.ds(tbase + RB, RB)], xrows[1],
                               isems[1])

    mstart = wid * (TPW // L)
    zeros = jnp.zeros((L,), jnp.int32)

    def scan_body(i, carry):
        counts, pre = carry
        pre = jnp.where(i == mstart, counts, pre)
        eid = eall[pl.ds(i * L, L)]
        for e in range(E):
            m = eid == e
            p = plsc.all_reduce_population_count(m)
            counts = counts + jnp.where(lane == e, p, 0)
        return counts, pre

    tot, pre = lax.fori_loop(0, NVREG, scan_body, (zeros, zeros), unroll=4)

    padded = ((tot + (TILE - 1)) // TILE) * TILE
    incl = plsc.cumsum(padded)
    off = incl - padded
    basevec = off + pre

    def lane_scalar(vec, e):
        return jnp.sum(jnp.where(lane == e, vec, 0))

    bases = [lane_scalar(basevec, e) for e in range(E)]

    pair0 = wid * TPW
    for v in range(TPW // L):
        eid = eall[pl.ds(pair0 + v * L, L)]
        pos = zeros
        for e in range(E):
            m = eid == e
            mi = m.astype(jnp.int32)
            rank = plsc.cumsum(mi) - 1
            pos = jnp.where(m, rank + bases[e], pos)
            bases[e] = bases[e] + jnp.sum(mi)
        pbufs = (pb0, pb1, pb2, pb3, pb4, pb5, pb6, pb7)
        pbufs[v // (RB // L)][pl.ds((v % (RB // L)) * L, L)] = pos
        posflat[pl.ds(v * L, L)] = pos

    @pl.when(wid < 16)
    def _():
        pltpu.async_copy(posflat, pos0_hbm.at[pl.ds(tbase, TPW)],
                         psem).wait()

    @pl.when(wid >= 16)
    def _():
        pltpu.async_copy(posflat, pos1_hbm.at[pl.ds(tbase, TPW)],
                         psem).wait()

    nb = TPW // RB
    pbufs = (pb0, pb1, pb2, pb3, pb4, pb5, pb6, pb7)
    wrows = (wrowA, wrowB)
    xsems = (xsemA, xsemB)
    wsems = (wsemA, wsemB)
    xcp = [None, None]
    wcp = [None, None]
    for b in range(nb):
        s = b % 2
        incp[s].wait()
        for g in range(RB // L):
            wv = wvals[pl.ds(b * RB + g * L, L)]
            for r in range(L):
                wrows[s][g * L + r, pl.ds(0, L)] = jnp.broadcast_to(wv[r],
                                                                    (L,))
        xcp[s] = pltpu.async_copy(xrows[s], xg_hbm.at[pbufs[b]], xsems[s])
        wcp[s] = pltpu.async_copy(wrows[s], rw_hbm.at[pbufs[b]], wsems[s])
        if b + 2 < nb:
            xcp[s].wait()
            wcp[s].wait()
            incp[s] = pltpu.async_copy(
                x_hbm.at[pl.ds(tbase + (b + 2) * RB, RB)], xrows[s], isems[s])
    xcp[0].wait()
    wcp[0].wait()
    xcp[1].wait()
    wcp[1].wait()

    @pl.when(wid == 0)
    def _():
        total_padded = jnp.sum(padded)
        n_used = total_padded // TILE
        last_e = jnp.max(jnp.where(padded > 0, lane, -1))
        offs = [lane_scalar(off, e) for e in range(E)]
        for vi in range(4):
            ts = (jnp.arange(L, dtype=jnp.int32) + vi * L) * TILE
            te = jnp.full((L,), -1, jnp.int32)
            for e in range(E):
                te = te + jnp.where(ts >= offs[e], 1, 0)
            te = jnp.where(ts >= total_padded, last_e, te)
            tebuf[pl.ds(vi * L, L)] = te
        nubuf[...] = jnp.broadcast_to(n_used, (L,)).astype(jnp.int32)
        pltpu.sync_copy(tebuf, te_hbm)
        pltpu.sync_copy(nubuf, nu_hbm)


def _route(e0, e1, w0, w1, x2d):
    mesh = plsc.VectorSubcoreMesh(core_axis_name="c", subcore_axis_name="s")
    k = pl.kernel(
        _route_body,
        out_type=[
            jax.ShapeDtypeStruct((T,), jnp.int32),
            jax.ShapeDtypeStruct((T,), jnp.int32),
            jax.ShapeDtypeStruct((NPAD, D), jnp.float32),
            jax.ShapeDtypeStruct((NPAD, 128), jnp.float32),
            jax.ShapeDtypeStruct((64,), jnp.int32),
            jax.ShapeDtypeStruct((L,), jnp.int32),
        ],
        mesh=mesh,
        compiler_params=pltpu.CompilerParams(needs_layout_passes=False),
        scratch_types=(
            [pltpu.VMEM((NPAIR,), jnp.int32)]
            + [pltpu.VMEM((RB,), jnp.int32)] * 8
            + [pltpu.VMEM((TPW,), jnp.int32)]
            + [pltpu.VMEM((TPW,), jnp.float32)]
            + [pltpu.VMEM((RB, 128), jnp.float32)] * 2
            + [pltpu.VMEM((RB, D), jnp.float32)] * 2
            + [pltpu.VMEM((64,), jnp.int32),
               pltpu.VMEM((L,), jnp.int32)]
            + [pltpu.SemaphoreType.DMA] * 7
        ),
    )
    return k(e0, e1, w0, w1, x2d)


def _ffn_body(te_ref, nu_ref, xg_ref, w1_ref, w2_ref, rw_ref, o_ref):
    i = pl.program_id(0)

    @pl.when(i < nu_ref[0])
    def _():
        xt = xg_ref[...]
        h = jnp.dot(xt, w1_ref[0], preferred_element_type=jnp.float32)
        h = jax.nn.gelu(h)
        y = jnp.dot(h, w2_ref[0], preferred_element_type=jnp.float32)
        o_ref[...] = y * rw_ref[...][:, 0:1]


def _ffn(te, nu, xg, w1, w2, rw):
    grid_spec = pltpu.PrefetchScalarGridSpec(
        num_scalar_prefetch=2,
        grid=(NT,),
        in_specs=[
            pl.BlockSpec((TILE, D), lambda i, te, nu: (i, 0)),
            pl.BlockSpec((1, D, F), lambda i, te, nu: (te[i], 0, 0)),
            pl.BlockSpec((1, F, D), lambda i, te, nu: (te[i], 0, 0)),
            pl.BlockSpec((TILE, 128), lambda i, te, nu: (i, 0)),
        ],
        out_specs=pl.BlockSpec((TILE, D), lambda i, te, nu: (i, 0)),
    )
    return pl.pallas_call(
        _ffn_body,
        grid_spec=grid_spec,
        out_shape=jax.ShapeDtypeStruct((NPAD, D), jnp.float32),
        compiler_params=pltpu.CompilerParams(
            dimension_semantics=("arbitrary",),
        ),
    )(te, nu, xg, w1, w2, rw)


CB = 16


def _combine_body(yg_hbm, p0_hbm, p1_hbm, out_hbm, p0b, p1b,
                  y0A, y0B, y1A, y1B,
                  g0A, g0B, g1A, g1B, osA, osB):
    cid = lax.axis_index("c")
    sid = lax.axis_index("s")
    wid = sid * 2 + cid
    tpw = T // NW
    tbase = wid * tpw
    nb = tpw // CB

    pltpu.sync_copy(p0_hbm.at[pl.ds(tbase, tpw)], p0b)
    pltpu.sync_copy(p1_hbm.at[pl.ds(tbase, tpw)], p1b)

    y0s = (y0A, y0B)
    y1s = (y1A, y1B)
    g0s = (g0A, g0B)
    g1s = (g1A, g1B)
    oss = (osA, osB)

    def idx(pb, b):
        return pb[pl.ds(b * CB, CB)]

    cp0 = [None, None]
    cp1 = [None, None]
    wcp = [None, None]
    cp0[0] = pltpu.async_copy(yg_hbm.at[idx(p0b, 0)], y0s[0], g0s[0])
    cp1[0] = pltpu.async_copy(yg_hbm.at[idx(p1b, 0)], y1s[0], g1s[0])
    for b in range(nb):
        s = b % 2
        n = (b + 1) % 2
        if b + 1 < nb:
            if wcp[n] is not None:
                wcp[n].wait()
            cp0[n] = pltpu.async_copy(yg_hbm.at[idx(p0b, b + 1)], y0s[n],
                                      g0s[n])
            cp1[n] = pltpu.async_copy(yg_hbm.at[idx(p1b, b + 1)], y1s[n],
                                      g1s[n])
        cp0[s].wait()
        cp1[s].wait()

        def row_add(r, _, s=s):
            def d_add(dd, __):
                y0s[s][r, pl.ds(dd * L, L)] = (y0s[s][r, pl.ds(dd * L, L)]
                                               + y1s[s][r, pl.ds(dd * L, L)])
                return __
            return lax.fori_loop(0, D // L, d_add, _, unroll=16)

        lax.fori_loop(0, CB, row_add, 0, unroll=4)
        wcp[s] = pltpu.async_copy(y0s[s], out_hbm.at[pl.ds(tbase + b * CB,
                                                           CB)], oss[s])
    wcp[0].wait()
    wcp[1].wait()


def _combine(yg, pos0, pos1):
    mesh = plsc.VectorSubcoreMesh(core_axis_name="c", subcore_axis_name="s")
    nb = (T // NW) // RB
    k = pl.kernel(
        _combine_body,
        out_type=jax.ShapeDtypeStruct((T, D), jnp.float32),
        mesh=mesh,
        compiler_params=pltpu.CompilerParams(needs_layout_passes=False),
        scratch_types=(
            [pltpu.VMEM((T // NW,), jnp.int32)] * 2
            + [pltpu.VMEM((CB, D), jnp.float32)] * 4
            + [pltpu.SemaphoreType.DMA] * 6
        ),
    )
    return k(yg, pos0, pos1)


@jax.jit
def kernel(x, Wr, W1, W2):
    x2d = x.reshape(T, D)
    wr_pad = jnp.pad(Wr, ((0, 0), (0, 128 - E)))
    e0b, e1b, w0b, w1b = _router(x2d, wr_pad)
    e0 = e0b[:, 0]
    e1 = e1b[:, 0]
    w0 = w0b[:, 0]
    w1 = w1b[:, 0]
    pos0, pos1, xg, rw, te, nu = _route(e0, e1, w0, w1, x2d)
    yg = _ffn(te, nu, xg, W1, W2, rw)
    out = _combine(yg, pos0, pos1)
    return out.reshape(B, S, D)
